# baseline (speedup 1.0000x reference)
import functools

import jax
import jax.numpy as jnp
from jax import lax
from jax.experimental import pallas as pl
from jax.experimental.pallas import tpu as pltpu
from jax.experimental.pallas import tpu_sc as plsc

_NUM_EXPERTS = 64
_TOP_K = 8
_N = 32768


_BLOCK_COLS = 2048


def _softmax_body(rand_ref, full_ref):
    rand = rand_ref[...]
    m = jnp.max(rand, axis=0, keepdims=True)
    e = jnp.exp(rand - m)
    full_ref[...] = e / jnp.sum(e, axis=0, keepdims=True)


def _tc_full_gates_t(random_t):
    num_experts, n = random_t.shape
    return pl.pallas_call(
        _softmax_body,
        grid=(n // _BLOCK_COLS,),
        in_specs=[pl.BlockSpec((num_experts, _BLOCK_COLS), lambda i: (0, i))],
        out_specs=pl.BlockSpec((num_experts, _BLOCK_COLS), lambda i: (0, i)),
        out_shape=jax.ShapeDtypeStruct((num_experts, n), jnp.float32),
    )(random_t)


_NW = 32
_NODES_PER_W = _N // _NW
_B = 256
_BP = _B + 1
_NBLK = _NODES_PER_W // _B


def _sc_router_body(perm_hbm, tg_hbm, sparse_hbm, idx_hbm,
                    perm_buf, tg_buf, out_buf, idx_buf):
    wid = lax.axis_index("s") * 2 + lax.axis_index("c")
    w_base = wid * _NODES_PER_W

    iota = lax.iota(jnp.int32, 16)
    lo8 = iota < 8
    col8 = jnp.bitwise_and(iota, 7)
    zeros16 = jnp.zeros((16,), jnp.float32)

    def merge(ak, av, bk, bv):
        kk = jnp.where(lo8, ak, lax.rev(bk, (0,)))
        vv = jnp.where(lo8, av, lax.rev(bv, (0,)))
        return plsc.sort_key_val(kk, vv)

    def zero_body(e):
        for k in range(_B // 16):
            out_buf[e, pl.ds(16 * k, 16)] = zeros16

    def node_body(c):
        cvec = jnp.broadcast_to(c, (16,))

        sk = []
        sv = []
        for j in range(4):
            k_j, v_j = plsc.sort_key_val(
                plsc.load_gather(perm_buf, [iota + 16 * j, cvec]),
                iota + 16 * j)
            sk.append(k_j)
            sv.append(v_j)
        m01k, m01v = merge(sk[0], sv[0], sk[1], sv[1])
        m23k, m23v = merge(sk[2], sv[2], sk[3], sv[3])
        _, fv = merge(m01k, m01v, m23k, m23v)

        gv = plsc.load_gather(tg_buf, [col8, cvec])
        e = jnp.exp(gv)
        s = jnp.sum(jnp.where(lo8, e, zeros16))
        tg = e / s

        plsc.store_scatter(out_buf, [fv, cvec], tg, mask=lo8)
        plsc.store_scatter(idx_buf, [col8, cvec], fv, mask=lo8)

    for blk in range(_NBLK):
        base = w_base + blk * _B
        pltpu.sync_copy(perm_hbm.at[:, pl.ds(base, _B)],
                        perm_buf.at[:, pl.ds(0, _B)])
        pltpu.sync_copy(tg_hbm.at[:, pl.ds(base, _B)],
                        tg_buf.at[:, pl.ds(0, _B)])
        plsc.parallel_loop(0, _NUM_EXPERTS, 1, unroll=4)(zero_body)
        plsc.parallel_loop(0, _B, 1, unroll=8)(node_body)
        pltpu.sync_copy(out_buf.at[:, pl.ds(0, _B)],
                        sparse_hbm.at[:, pl.ds(base, _B)])
        pltpu.sync_copy(idx_buf.at[:, pl.ds(0, _B)],
                        idx_hbm.at[:, pl.ds(base, _B)])


_sc_router = functools.partial(
    pl.kernel,
    out_type=[
        jax.ShapeDtypeStruct((_NUM_EXPERTS, _N), jnp.float32),
        jax.ShapeDtypeStruct((_TOP_K, _N), jnp.int32),
    ],
    mesh=plsc.VectorSubcoreMesh(core_axis_name="c", subcore_axis_name="s"),
    compiler_params=pltpu.CompilerParams(needs_layout_passes=False),
    scratch_types=[
        pltpu.VMEM((_NUM_EXPERTS, _BP), jnp.float32),
        pltpu.VMEM((_TOP_K, _BP), jnp.float32),
        pltpu.VMEM((_NUM_EXPERTS, _BP), jnp.float32),
        pltpu.VMEM((_TOP_K, _BP), jnp.int32),
    ],
)(_sc_router_body)


@jax.jit
def kernel(random_raw, perm_raw, topk_gates_raw):
    full_gates_t = _tc_full_gates_t(random_raw.T)
    sparse_t, idx_t = _sc_router(perm_raw.T, topk_gates_raw.T)
    return (sparse_t.T, idx_t.T, full_gates_t.T)


# speedup vs baseline: 1.0330x; 1.0330x over previous
import functools

import jax
import jax.numpy as jnp
from jax import lax
from jax.experimental import pallas as pl
from jax.experimental.pallas import tpu as pltpu
from jax.experimental.pallas import tpu_sc as plsc

_NUM_EXPERTS = 64
_TOP_K = 8
_N = 32768


_BLOCK_COLS = 2048


def _softmax_body(rand_ref, full_ref):
    rand = rand_ref[...]
    m = jnp.max(rand, axis=0, keepdims=True)
    e = jnp.exp(rand - m)
    full_ref[...] = e / jnp.sum(e, axis=0, keepdims=True)


def _tc_full_gates_t(random_t):
    num_experts, n = random_t.shape
    return pl.pallas_call(
        _softmax_body,
        grid=(n // _BLOCK_COLS,),
        in_specs=[pl.BlockSpec((num_experts, _BLOCK_COLS), lambda i: (0, i))],
        out_specs=pl.BlockSpec((num_experts, _BLOCK_COLS), lambda i: (0, i)),
        out_shape=jax.ShapeDtypeStruct((num_experts, n), jnp.float32),
    )(random_t)


_NW = 32
_NODES_PER_W = _N // _NW
_B = 256
_BP = _B + 1
_NBLK = _NODES_PER_W // _B


def _sc_router_body(perm_hbm, tg_hbm, sparse_hbm, idx_hbm,
                    perm_buf, tg_buf, out_buf, idx_buf):
    wid = lax.axis_index("s") * 2 + lax.axis_index("c")
    w_base = wid * _NODES_PER_W

    iota = lax.iota(jnp.int32, 16)
    lo8 = iota < 8
    col8 = jnp.bitwise_and(iota, 7)
    zeros16 = jnp.zeros((16,), jnp.float32)

    def merge(ak, av, bk, bv):
        kk = jnp.where(lo8, ak, lax.rev(bk, (0,)))
        vv = jnp.where(lo8, av, lax.rev(bv, (0,)))
        return plsc.sort_key_val(kk, vv)

    def zero_body(e):
        for k in range(_B // 16):
            out_buf[e, pl.ds(16 * k, 16)] = zeros16

    def node_body(c):
        cvec = jnp.broadcast_to(c, (16,))

        sk = []
        sv = []
        for j in range(4):
            k_j, v_j = plsc.sort_key_val(
                plsc.load_gather(perm_buf, [iota + 16 * j, cvec]),
                iota + 16 * j)
            sk.append(k_j)
            sv.append(v_j)
        m01k, m01v = merge(sk[0], sv[0], sk[1], sv[1])
        m23k, m23v = merge(sk[2], sv[2], sk[3], sv[3])
        _, fv = merge(m01k, m01v, m23k, m23v)

        gv = plsc.load_gather(tg_buf, [col8, cvec])
        e = jnp.exp(gv)
        s = jnp.sum(jnp.where(lo8, e, zeros16))
        tg = e / s

        plsc.store_scatter(out_buf, [fv, cvec], tg, mask=lo8)
        plsc.store_scatter(idx_buf, [col8, cvec], fv, mask=lo8)

    for blk in range(_NBLK):
        base = w_base + blk * _B
        pltpu.sync_copy(perm_hbm.at[:, pl.ds(base, _B)],
                        perm_buf.at[:, pl.ds(0, _B)])
        pltpu.sync_copy(tg_hbm.at[:, pl.ds(base, _B)],
                        tg_buf.at[:, pl.ds(0, _B)])
        plsc.parallel_loop(0, _NUM_EXPERTS, 1, unroll=4)(zero_body)
        plsc.parallel_loop(0, _B, 1, unroll=4)(node_body)
        pltpu.sync_copy(out_buf.at[:, pl.ds(0, _B)],
                        sparse_hbm.at[:, pl.ds(base, _B)])
        pltpu.sync_copy(idx_buf.at[:, pl.ds(0, _B)],
                        idx_hbm.at[:, pl.ds(base, _B)])


_sc_router = functools.partial(
    pl.kernel,
    out_type=[
        jax.ShapeDtypeStruct((_NUM_EXPERTS, _N), jnp.float32),
        jax.ShapeDtypeStruct((_TOP_K, _N), jnp.int32),
    ],
    mesh=plsc.VectorSubcoreMesh(core_axis_name="c", subcore_axis_name="s"),
    compiler_params=pltpu.CompilerParams(needs_layout_passes=False),
    scratch_types=[
        pltpu.VMEM((_NUM_EXPERTS, _BP), jnp.float32),
        pltpu.VMEM((_TOP_K, _BP), jnp.float32),
        pltpu.VMEM((_NUM_EXPERTS, _BP), jnp.float32),
        pltpu.VMEM((_TOP_K, _BP), jnp.int32),
    ],
)(_sc_router_body)


@jax.jit
def kernel(random_raw, perm_raw, topk_gates_raw):
    full_gates_t = _tc_full_gates_t(random_raw.T)
    sparse_t, idx_t = _sc_router(perm_raw.T, topk_gates_raw.T)
    return (sparse_t.T, idx_t.T, full_gates_t.T)
